# Initial kernel scaffold; baseline (speedup 1.0000x reference)
import jax
import jax.numpy as jnp
from jax.experimental import pallas as pl


def kernel(input_ids, position_ids, text_attn_masks, word_table, pos_table, ln_gamma, ln_beta):
    raise NotImplementedError("write your pallas kernel here")


# speedup vs baseline: 2.6010x; 2.6010x over previous
import functools

import jax
import jax.numpy as jnp
from jax import lax
from jax.experimental import pallas as pl
from jax.experimental.pallas import tpu as pltpu
from jax.experimental.pallas import tpu_sc as plsc

VOCAB = 100000
HIDDEN = 128
MAX_POS = 512
B, L = 4096, 50
N = B * L
NC, NS = 2, 16
NW = NC * NS
PER_W = N // NW
C = 128
CHUNKS = PER_W // C
LANES = 8
EPS = 1e-12


def _rsqrt(v):
    i = lax.bitcast_convert_type(v, jnp.int32)
    i = jnp.int32(0x5F3759DF) - (i >> 1)
    y = lax.bitcast_convert_type(i, jnp.float32)
    for _ in range(3):
        y = y * (1.5 - 0.5 * v * y * y)
    return y


def _sc_embed_ln(word_table, pos_table, ids, pids, gamma, beta):
    mesh = plsc.VectorSubcoreMesh(core_axis_name="c", subcore_axis_name="s")

    @functools.partial(
        pl.kernel,
        mesh=mesh,
        out_type=jax.ShapeDtypeStruct((N, HIDDEN), jnp.float32),
        scratch_types=[
            pltpu.VMEM((C,), jnp.int32),
            pltpu.VMEM((C,), jnp.int32),
            pltpu.VMEM((C, HIDDEN), jnp.float32),
            pltpu.VMEM((C, HIDDEN), jnp.float32),
            pltpu.VMEM((HIDDEN,), jnp.float32),
            pltpu.VMEM((HIDDEN,), jnp.float32),
            pltpu.SemaphoreType.DMA,
        ],
    )
    def k(word_hbm, pos_hbm, ids_hbm, pids_hbm, gam_hbm, bet_hbm, out_hbm,
          ids_v, pids_v, rows_v, prows_v, gam_v, bet_v, sem):
        wid = lax.axis_index("s") * NC + lax.axis_index("c")
        base = wid * PER_W
        iota = lax.iota(jnp.int32, 16)

        def xsum(v):
            for d in (1, 2, 4, 8):
                v = v + v.at[iota ^ d].get(mode="promise_in_bounds")
            return v

        pltpu.sync_copy(gam_hbm, gam_v)
        pltpu.sync_copy(bet_hbm, bet_v)
        gam = [gam_v[pl.ds(16 * j, 16)] for j in range(LANES)]
        bet = [bet_v[pl.ds(16 * j, 16)] for j in range(LANES)]

        def chunk_body(ci, carry):
            off = base + ci * C
            pltpu.sync_copy(ids_hbm.at[pl.ds(off, C)], ids_v)
            pltpu.sync_copy(pids_hbm.at[pl.ds(off, C)], pids_v)
            pltpu.async_copy(word_hbm.at[ids_v], rows_v, sem).wait()
            pltpu.async_copy(pos_hbm.at[pids_v], prows_v, sem).wait()

            def tok_body(t, tc):
                x = [rows_v[t, pl.ds(16 * j, 16)] + prows_v[t, pl.ds(16 * j, 16)]
                     for j in range(LANES)]
                s = x[0]
                ss = x[0] * x[0]
                for j in range(1, LANES):
                    s = s + x[j]
                    ss = ss + x[j] * x[j]
                tot = xsum(s)
                tot2 = xsum(ss)
                mean = tot * (1.0 / HIDDEN)
                var = tot2 * (1.0 / HIDDEN) - mean * mean
                r = _rsqrt(var + EPS)
                for j in range(LANES):
                    rows_v[t, pl.ds(16 * j, 16)] = (x[j] - mean) * r * gam[j] + bet[j]
                return tc

            lax.fori_loop(0, C, tok_body, 0)
            pltpu.sync_copy(rows_v, out_hbm.at[pl.ds(off, C)])
            return carry

        lax.fori_loop(0, CHUNKS, chunk_body, 0)

    return k(word_table, pos_table, ids, pids, gamma, beta)


def kernel(input_ids, position_ids, text_attn_masks, word_table, pos_table,
           ln_gamma, ln_beta):
    ids = input_ids.reshape(-1).astype(jnp.int32)
    pids = position_ids.reshape(-1).astype(jnp.int32)
    out = _sc_embed_ln(word_table, pos_table, ids, pids, ln_gamma, ln_beta)
    return (out.reshape(B, L, HIDDEN), text_attn_masks)


# speedup vs baseline: 4.8369x; 1.8597x over previous
import functools

import jax
import jax.numpy as jnp
from jax import lax
from jax.experimental import pallas as pl
from jax.experimental.pallas import tpu as pltpu
from jax.experimental.pallas import tpu_sc as plsc

VOCAB = 100000
HIDDEN = 128
MAX_POS = 512
B, L = 4096, 50
N = B * L
NC, NS = 2, 16
NW = NC * NS
PER_W = N // NW
C = 128
CHUNKS = PER_W // C
LANES = 8
U = 4
EPS = 1e-12


def _rsqrt(v):
    i = lax.bitcast_convert_type(v, jnp.int32)
    i = jnp.int32(0x5F3759DF) - (i >> 1)
    y = lax.bitcast_convert_type(i, jnp.float32)
    for _ in range(2):
        y = y * (1.5 - 0.5 * v * y * y)
    return y


def _sc_embed_ln(word_table, pos_table, ids, pids, gamma, beta):
    mesh = plsc.VectorSubcoreMesh(core_axis_name="c", subcore_axis_name="s")

    @functools.partial(
        pl.kernel,
        mesh=mesh,
        out_type=jax.ShapeDtypeStruct((N, HIDDEN), jnp.float32),
        scratch_types=[
            pltpu.VMEM((C,), jnp.int32),
            pltpu.VMEM((C,), jnp.int32),
            pltpu.VMEM((C,), jnp.int32),
            pltpu.VMEM((C,), jnp.int32),
            pltpu.VMEM((C, HIDDEN), jnp.float32),
            pltpu.VMEM((C, HIDDEN), jnp.float32),
            pltpu.VMEM((C, HIDDEN), jnp.float32),
            pltpu.VMEM((C, HIDDEN), jnp.float32),
            pltpu.VMEM((C, HIDDEN), jnp.float32),
            pltpu.VMEM((C, HIDDEN), jnp.float32),
            pltpu.VMEM((HIDDEN,), jnp.float32),
            pltpu.VMEM((HIDDEN,), jnp.float32),
            pltpu.SemaphoreType.DMA,
            pltpu.SemaphoreType.DMA,
            pltpu.SemaphoreType.DMA,
            pltpu.SemaphoreType.DMA,
            pltpu.SemaphoreType.DMA,
            pltpu.SemaphoreType.DMA,
        ],
    )
    def k(word_hbm, pos_hbm, ids_hbm, pids_hbm, gam_hbm, bet_hbm, out_hbm,
          ids0, ids1, pids0, pids1, rows0, rows1, prows0, prows1, ob0, ob1,
          gam_v, bet_v, isem0, isem1, gsem0, gsem1, wsem0, wsem1):
        wid = lax.axis_index("s") * NC + lax.axis_index("c")
        base = wid * PER_W
        pltpu.sync_copy(gam_hbm, gam_v)
        pltpu.sync_copy(bet_hbm, bet_v)
        gam = [gam_v[pl.ds(16 * j, 16)] for j in range(LANES)]
        bet = [bet_v[pl.ds(16 * j, 16)] for j in range(LANES)]
        iota = lax.iota(jnp.int32, 16)

        def xsum(v):
            for d in (1, 2, 4, 8):
                v = v + v.at[iota ^ d].get(mode="promise_in_bounds")
            return v

        def compute(rows_b, prows_b, ob_b):
            def tok_body(ti, tc):
                for u in range(U):
                    t = ti * U + u
                    x = [rows_b[t, pl.ds(16 * j, 16)] +
                         prows_b[t, pl.ds(16 * j, 16)] for j in range(LANES)]
                    s = x[0]
                    ss = x[0] * x[0]
                    for j in range(1, LANES):
                        s = s + x[j]
                        ss = ss + x[j] * x[j]
                    mean = xsum(s) * (1.0 / HIDDEN)
                    var = xsum(ss) * (1.0 / HIDDEN) - mean * mean
                    r = _rsqrt(var + EPS)
                    for j in range(LANES):
                        ob_b[t, pl.ds(16 * j, 16)] = \
                            (x[j] - mean) * r * gam[j] + bet[j]
                return tc
            lax.fori_loop(0, C // U, tok_body, 0)

        def chunk_step(ci, bufs):
            ids_b, pids_b, rows_b, prows_b, ob_b, isem, gsem, wsem = bufs
            off = base + ci * C
            pltpu.make_async_copy(word_hbm.at[ids_b], rows_b, gsem).wait()
            pltpu.make_async_copy(pos_hbm.at[pids_b], prows_b, gsem).wait()

            @pl.when(ci >= 2)
            def _():
                pltpu.make_async_copy(
                    ob_b, out_hbm.at[pl.ds(off - 2 * C, C)], wsem).wait()

            @pl.when(ci + 2 < CHUNKS)
            def _():
                off2 = off + 2 * C
                pltpu.make_async_copy(
                    ids_hbm.at[pl.ds(off2, C)], ids_b, isem).start()
                pltpu.make_async_copy(
                    pids_hbm.at[pl.ds(off2, C)], pids_b, isem).start()

            compute(rows_b, prows_b, ob_b)
            pltpu.make_async_copy(ob_b, out_hbm.at[pl.ds(off, C)], wsem).start()

            @pl.when(ci + 2 < CHUNKS)
            def _():
                off2 = off + 2 * C
                pltpu.make_async_copy(
                    ids_hbm.at[pl.ds(off2, C)], ids_b, isem).wait()
                pltpu.make_async_copy(
                    pids_hbm.at[pl.ds(off2, C)], pids_b, isem).wait()
                pltpu.make_async_copy(word_hbm.at[ids_b], rows_b, gsem).start()
                pltpu.make_async_copy(pos_hbm.at[pids_b], prows_b, gsem).start()

        slot0 = (ids0, pids0, rows0, prows0, ob0, isem0, gsem0, wsem0)
        slot1 = (ids1, pids1, rows1, prows1, ob1, isem1, gsem1, wsem1)

        pltpu.sync_copy(ids_hbm.at[pl.ds(base, C)], ids0)
        pltpu.sync_copy(pids_hbm.at[pl.ds(base, C)], pids0)
        pltpu.sync_copy(ids_hbm.at[pl.ds(base + C, C)], ids1)
        pltpu.sync_copy(pids_hbm.at[pl.ds(base + C, C)], pids1)
        pltpu.make_async_copy(word_hbm.at[ids0], rows0, gsem0).start()
        pltpu.make_async_copy(pos_hbm.at[pids0], prows0, gsem0).start()
        pltpu.make_async_copy(word_hbm.at[ids1], rows1, gsem1).start()
        pltpu.make_async_copy(pos_hbm.at[pids1], prows1, gsem1).start()

        def pair_body(p, carry):
            chunk_step(2 * p, slot0)
            chunk_step(2 * p + 1, slot1)
            return carry

        lax.fori_loop(0, CHUNKS // 2, pair_body, 0)

        endo = base + (CHUNKS - 2) * C
        pltpu.make_async_copy(ob0, out_hbm.at[pl.ds(endo, C)], wsem0).wait()
        pltpu.make_async_copy(ob1, out_hbm.at[pl.ds(endo + C, C)], wsem1).wait()

    return k(word_table, pos_table, ids, pids, gamma, beta)


def kernel(input_ids, position_ids, text_attn_masks, word_table, pos_table,
           ln_gamma, ln_beta):
    ids = input_ids.reshape(-1).astype(jnp.int32)
    pids = position_ids.reshape(-1).astype(jnp.int32)
    out = _sc_embed_ln(word_table, pos_table, ids, pids, ln_gamma, ln_beta)
    return (out.reshape(B, L, HIDDEN), text_attn_masks)
